# baseline (speedup 1.0000x reference)
import functools

import jax
import jax.numpy as jnp
from jax import lax
from jax.experimental import pallas as pl
from jax.experimental.pallas import tpu as pltpu
from jax.experimental.pallas import tpu_sc as plsc


def _fps(pts_cbn, npoint):
    _, B, N = pts_cbn.shape

    def kern(p_ref, o_ref, dists, far):
        i = pl.program_id(0)

        @pl.when(i == 0)
        def _():
            dists[...] = jnp.full((B, N), 1e10, jnp.float32)
            far[...] = jnp.zeros((B, 1), jnp.int32)

        f = far[...]
        o_ref[0] = f
        x = p_ref[0]
        y = p_ref[1]
        z = p_ref[2]
        iota = lax.broadcasted_iota(jnp.int32, (B, N), 1)
        sel = iota == f
        cx = jnp.sum(jnp.where(sel, x, 0.0), axis=1, keepdims=True)
        cy = jnp.sum(jnp.where(sel, y, 0.0), axis=1, keepdims=True)
        cz = jnp.sum(jnp.where(sel, z, 0.0), axis=1, keepdims=True)
        d = (x - cx) ** 2 + (y - cy) ** 2 + (z - cz) ** 2
        nd = jnp.minimum(dists[...], d)
        dists[...] = nd
        m = jnp.max(nd, axis=1, keepdims=True)
        cand = jnp.where(nd == m, iota, N)
        far[...] = jnp.min(cand, axis=1, keepdims=True)

    out = pl.pallas_call(
        kern,
        grid=(npoint,),
        in_specs=[pl.BlockSpec((3, B, N), lambda i: (0, 0, 0))],
        out_specs=pl.BlockSpec((1, B, 1), lambda i: (i, 0, 0)),
        out_shape=jax.ShapeDtypeStruct((npoint, B, 1), jnp.int32),
        scratch_shapes=[pltpu.VMEM((B, N), jnp.float32),
                        pltpu.VMEM((B, 1), jnp.int32)],
    )(pts_cbn)
    return out[:, :, 0].T


def _bf(x):
    return x.astype(jnp.bfloat16).astype(jnp.float32)


def _dist_parts(q, p):
    q0, q1, q2 = q[:, 0:1], q[:, 1:2], q[:, 2:3]
    p0, p1, p2 = p[0:1, :], p[1:2, :], p[2:3, :]
    cross = (_bf(q0) * _bf(p0) + _bf(q1) * _bf(p1) + _bf(q2) * _bf(p2))
    qn = q0 * q0 + q1 * q1 + q2 * q2
    pn = p0 * p0 + p1 * p1 + p2 * p2
    return (qn - 2.0 * cross) + pn


def _ballq(q_b, p_cn, r2, S):
    B, Q, _ = q_b.shape
    N = p_cn.shape[2]
    BQ = min(Q, 256)

    def kern(q_ref, p_ref, o_ref):
        b = pl.program_id(0)
        d = _dist_parts(q_ref[0], p_ref[0])
        iota = lax.broadcasted_iota(jnp.int32, (BQ, N), 1)
        keys = jnp.where(d <= r2, iota, iota + N)
        cols = []
        first = None
        for s in range(S):
            m = jnp.min(keys, axis=1, keepdims=True)
            if s == 0:
                first = jnp.where(m < N, m, 0)
                selc = first
            else:
                selc = jnp.where(m < N, m, first)
            cols.append(selc)
            keys = jnp.where(keys == m, 2 * N, keys)
        o_ref[0] = jnp.concatenate(cols, axis=1) + b * N

    return pl.pallas_call(
        kern,
        grid=(B, Q // BQ),
        in_specs=[pl.BlockSpec((1, BQ, 3), lambda b, j: (b, j, 0)),
                  pl.BlockSpec((1, 3, N), lambda b, j: (b, 0, 0))],
        out_specs=pl.BlockSpec((1, BQ, S), lambda b, j: (b, j, 0)),
        out_shape=jax.ShapeDtypeStruct((B, Q, S), jnp.int32),
    )(q_b, p_cn)


def _knn(q_b, p_cn, k):
    B, Q, _ = q_b.shape
    N = p_cn.shape[2]
    BQ = min(Q, 512)

    def kern(q_ref, p_ref, od_ref, oi_ref):
        b = pl.program_id(0)
        d = _dist_parts(q_ref[0], p_ref[0])
        iota = lax.broadcasted_iota(jnp.int32, (BQ, N), 1)
        dcols, icols = [], []
        for _ in range(k):
            m = jnp.min(d, axis=1, keepdims=True)
            cand = jnp.where(d == m, iota, N)
            selc = jnp.min(cand, axis=1, keepdims=True)
            dcols.append(m)
            icols.append(selc + b * N)
            d = jnp.where(iota == selc, jnp.float32(1e30), d)
        od_ref[0] = jnp.concatenate(dcols, axis=1)
        oi_ref[0] = jnp.concatenate(icols, axis=1)

    return pl.pallas_call(
        kern,
        grid=(B, Q // BQ),
        in_specs=[pl.BlockSpec((1, BQ, 3), lambda b, j: (b, j, 0)),
                  pl.BlockSpec((1, 3, N), lambda b, j: (b, 0, 0))],
        out_specs=[pl.BlockSpec((1, BQ, k), lambda b, j: (b, j, 0)),
                   pl.BlockSpec((1, BQ, k), lambda b, j: (b, j, 0))],
        out_shape=[jax.ShapeDtypeStruct((B, Q, k), jnp.float32),
                   jax.ShapeDtypeStruct((B, Q, k), jnp.int32)],
    )(q_b, p_cn)


def _mlp(x, layers, *, c=None, pre_relu=False, pool=None, S=1, wd=None, BM=512):
    M, C0 = x.shape
    BM = min(BM, M)
    nblk = M // BM
    CL = layers[-1][0].shape[0] if layers else C0
    Mout, BMout = (M // S, BM // S) if pool else (M, BM)

    args = [x]
    in_specs = [pl.BlockSpec((BM, C0), lambda i: (i, 0))]
    if c is not None:
        args.append(c)
        in_specs.append(pl.BlockSpec((BM, C0), lambda i: (i, 0)))
    if wd is not None:
        args.append(wd)
        in_specs.append(pl.BlockSpec((BMout, S), lambda i: (i, 0)))
    for (W, b, _r) in layers:
        Wa = W.T if W.shape[1] <= 8 else W.astype(jnp.bfloat16)
        args.append(Wa)
        in_specs.append(pl.BlockSpec(Wa.shape, lambda i: (0, 0)))
        args.append(b.reshape(1, -1))
        in_specs.append(pl.BlockSpec((1, b.shape[0]), lambda i: (0, 0)))

    def kern(*refs):
        it = iter(refs)
        h = next(it)[...]
        if c is not None:
            h = h + next(it)[...]
        if pre_relu:
            h = jnp.maximum(h, 0.0)
        wd_v = next(it)[...] if wd is not None else None
        for (W, _b, r) in layers:
            Wv = next(it)[...]
            bv = next(it)[...]
            if W.shape[1] <= 8:
                acc = bv
                for cc in range(W.shape[1]):
                    acc = acc + h[:, cc:cc + 1] * Wv[cc:cc + 1, :]
                h = acc
            else:
                h = lax.dot_general(h.astype(jnp.bfloat16), Wv,
                                    (((1,), (1,)), ((), ())),
                                    preferred_element_type=jnp.float32) + bv
            if r:
                h = jnp.maximum(h, 0.0)
        o_ref = next(it)
        if pool == 'max':
            h = jnp.max(h.reshape(BMout, S, CL), axis=1)
        elif pool == 'wsum':
            w = 1.0 / jnp.maximum(wd_v, 1e-10)
            w = w / jnp.sum(w, axis=1, keepdims=True)
            h = jnp.sum(h.reshape(BMout, S, CL) * w[:, :, None], axis=1)
        o_ref[...] = h

    return pl.pallas_call(
        kern,
        grid=(nblk,),
        in_specs=in_specs,
        out_specs=pl.BlockSpec((BMout, CL), lambda i: (i, 0)),
        out_shape=jax.ShapeDtypeStruct((Mout, CL), jnp.float32),
    )(*args)


def _sc_gather_impl(table, idx):
    _T, C = table.shape
    M = idx.shape[0]
    info = plsc.get_sparse_core_info()
    NC = info.num_cores
    NW = NC * info.num_subcores
    bpw = M // NW
    CH = 128 if bpw % 128 == 0 else bpw
    nch = bpw // CH
    mesh = plsc.VectorSubcoreMesh(core_axis_name="c", subcore_axis_name="s")

    @functools.partial(
        pl.kernel, mesh=mesh,
        compiler_params=pltpu.CompilerParams(use_tc_tiling_on_sc=False),
        out_type=jax.ShapeDtypeStruct((M, C), jnp.float32),
        scratch_types=[pltpu.VMEM((CH,), jnp.int32),
                       pltpu.VMEM((CH, C), jnp.float32),
                       pltpu.SemaphoreType.DMA],
    )
    def k(table_hbm, idx_hbm, out_hbm, idx_v, rows_v, sem):
        wid = lax.axis_index("s") * NC + lax.axis_index("c")
        base = wid * bpw

        def chunk(off):
            pltpu.sync_copy(idx_hbm.at[pl.ds(off, CH)], idx_v)
            pltpu.async_copy(table_hbm.at[idx_v], rows_v, sem).wait()
            pltpu.sync_copy(rows_v, out_hbm.at[pl.ds(off, CH)])

        if nch == 1:
            chunk(base)
        else:
            def body(j, carry):
                chunk(base + j * CH)
                return carry
            lax.fori_loop(0, nch, body, 0)

    return k(table, idx)


def _gather_rows(table, idx):
    M = idx.shape[0]
    C = table.shape[1]
    Cp = ((C + 15) // 16) * 16
    if Cp != C:
        table = jnp.pad(table, ((0, 0), (0, Cp - C)))
    Mp = ((M + 255) // 256) * 256
    if Mp != M:
        idx = jnp.concatenate([idx, jnp.zeros((Mp - M,), jnp.int32)])
    out = _sc_gather_impl(table, idx)
    return out[:M, :C]


def _set_conv(pts_cn, feats, npoint, r2, S, params):
    B, _, N = pts_cn.shape
    (W1, b1), (W2, b2), (W3, b3) = params
    idx = _fps(jnp.transpose(pts_cn, (1, 0, 2)), npoint)
    gidx = (idx + jnp.arange(B, dtype=jnp.int32)[:, None] * N).reshape(-1)
    ptsT = jnp.transpose(pts_cn, (0, 2, 1))
    ptab = jnp.pad(ptsT.reshape(B * N, 3), ((0, 0), (0, 13)))
    new_flat = _gather_rows(ptab, gidx)[:, :3]
    new_b = new_flat.reshape(B, npoint, 3)
    nidx = _ballq(new_b, pts_cn, r2, S)
    src = jnp.concatenate([ptsT, feats], axis=-1).reshape(B * N, -1)
    A = _mlp(src, [(W1, jnp.zeros_like(b1), False)])
    cvec = _mlp(new_flat, [(-W1[:, :3], b1, False)])
    G = _gather_rows(A, nidx.reshape(-1))
    crep = jnp.repeat(cvec, S, axis=0)
    H = _mlp(G, [(W2, b2, True), (W3, b3, True)],
             c=crep, pre_relu=True, pool='max', S=S)
    return jnp.transpose(new_b, (0, 2, 1)), new_b, H.reshape(B, npoint, -1)


def _flow_embedding(p1_cn, p1_b, p2_cn, f1, f2, S, params):
    B, Q, _ = p1_b.shape
    N = p2_cn.shape[2]
    (W1, b1), (W2, b2), (W3, b3) = params
    Cf = f2.shape[-1]
    _, nidx = _knn(p1_b, p2_cn, S)
    src = jnp.concatenate([jnp.transpose(p2_cn, (0, 2, 1)), f2],
                          axis=-1).reshape(B * N, -1)
    A = _mlp(src, [(W1[:, :3 + Cf], jnp.zeros_like(b1), False)])
    qsrc = jnp.concatenate([p1_b, f1], axis=-1).reshape(B * Q, -1)
    Wc = jnp.concatenate([-W1[:, :3], W1[:, 3 + Cf:]], axis=1)
    cvec = _mlp(qsrc, [(Wc, b1, False)])
    G = _gather_rows(A, nidx.reshape(-1))
    crep = jnp.repeat(cvec, S, axis=0)
    H = _mlp(G, [(W2, b2, True), (W3, b3, True)],
             c=crep, pre_relu=True, pool='max', S=S)
    return H.reshape(B, Q, -1)


def _set_upconv(pc_cn, pf_b, fc, ff, S, params1, params2):
    B, Nf, _ = pf_b.shape
    Nc = pc_cn.shape[2]
    Cc = fc.shape[-1]
    _, nidx = _knn(pf_b, pc_cn, S)
    pf_flat = pf_b.reshape(B * Nf, 3)
    pcT = jnp.transpose(pc_cn, (0, 2, 1))
    if params1:
        (W1, b1), (W2, b2), (W3, b3) = params1
        src = jnp.concatenate([pcT, fc], axis=-1).reshape(B * Nc, -1)
        A = _mlp(src, [(W1, jnp.zeros_like(b1), False)])
        cvec = _mlp(pf_flat, [(-W1[:, :3], b1, False)])
        G = _gather_rows(A, nidx.reshape(-1))
        crep = jnp.repeat(cvec, S, axis=0)
        H = _mlp(G, [(W2, b2, True), (W3, b3, True)],
                 c=crep, pre_relu=True, pool='max', S=S)
        lay2 = params2
    else:
        tab = jnp.concatenate([jnp.pad(pcT, ((0, 0), (0, 0), (0, 13))), fc],
                              axis=-1).reshape(B * Nc, -1)
        G = _gather_rows(tab, nidx.reshape(-1))
        cpad = jnp.pad(-pf_flat, ((0, 0), (0, 13 + Cc)))
        crep = jnp.repeat(cpad, S, axis=0)
        H = _mlp(G, [], c=crep, pool='max', S=S)
        (W1, b1) = params2[0]
        Z = jnp.zeros((W1.shape[0], 13), W1.dtype)
        W1m = jnp.concatenate([W1[:, :3], Z, W1[:, 3:]], axis=1)
        lay2 = [(W1m, b1)] + list(params2[1:])
    g2 = jnp.concatenate([H, ff.reshape(B * Nf, -1)], axis=-1)
    out = _mlp(g2, [(W, b, True) for (W, b) in lay2])
    return out.reshape(B, Nf, -1)


def _feature_prop_cls(pc_cn, pf_b, fc, ff, params_fp, params_cls):
    B, Nf, _ = pf_b.shape
    Nc = pc_cn.shape[2]
    d, nidx = _knn(pf_b, pc_cn, 3)
    G = _gather_rows(fc.reshape(B * Nc, -1), nidx.reshape(-1))
    interp = _mlp(G, [], pool='wsum', S=3,
                  wd=d.reshape(B * Nf, 3), BM=384)
    x = jnp.concatenate([interp, ff.reshape(B * Nf, -1)], axis=-1)
    (Wf1, bf1), (Wf2, bf2) = params_fp
    (Wc1, bc1), (Wc2, bc2) = params_cls
    out = _mlp(x, [(Wf1, bf1, True), (Wf2, bf2, True),
                   (Wc1, bc1, True), (Wc2, bc2, False)])
    return out.reshape(B, Nf, -1)


def kernel(points1, points2, features1, features2, params):
    B = points1.shape[0]
    f1 = jnp.transpose(features1, (0, 2, 1))
    f2 = jnp.transpose(features2, (0, 2, 1))

    p1_1cn, p1_1b, f1_1 = _set_conv(points1, f1, 1024, 0.25, 16, params['sc1'])
    p1_2cn, p1_2b, f1_2 = _set_conv(p1_1cn, f1_1, 256, 1.0, 16, params['sc2'])
    p2_1cn, _p2_1b, f2_1 = _set_conv(points2, f2, 1024, 0.25, 16, params['sc1'])
    p2_2cn, _p2_2b, f2_2 = _set_conv(p2_1cn, f2_1, 256, 1.0, 16, params['sc2'])

    emb = _flow_embedding(p1_2cn, p1_2b, p2_2cn, f1_2, f2_2, 64, params['fe'])

    p1_3cn, p1_3b, f1_3 = _set_conv(p1_2cn, emb, 64, 4.0, 8, params['sc3'])
    p1_4cn, _p1_4b, f1_4 = _set_conv(p1_3cn, f1_3, 16, 16.0, 8, params['sc4'])

    nf1_3 = _set_upconv(p1_4cn, p1_3b, f1_4, f1_3, 8,
                        params['up1_1'], params['up1_2'])
    nf1_2 = _set_upconv(p1_3cn, p1_2b, nf1_3,
                        jnp.concatenate([f1_2, emb], axis=-1), 8,
                        params['up2_1'], params['up2_2'])
    nf1_1 = _set_upconv(p1_2cn, p1_1b, nf1_2, f1_1, 8,
                        params['up3_1'], params['up3_2'])

    p1b = jnp.transpose(points1, (0, 2, 1))
    flow = _feature_prop_cls(p1_1cn, p1b, nf1_1, f1,
                             params['fp'], params['cls'])
    return jnp.transpose(flow, (0, 2, 1))


# speedup vs baseline: 1.4424x; 1.4424x over previous
import functools

import jax
import jax.numpy as jnp
from jax import lax
from jax.experimental import pallas as pl
from jax.experimental.pallas import tpu as pltpu
from jax.experimental.pallas import tpu_sc as plsc


def _fps(pts_cbn, npoint):
    _, B, N = pts_cbn.shape

    def kern(p_ref, o_ref, dists, far):
        i = pl.program_id(0)

        @pl.when(i == 0)
        def _():
            dists[...] = jnp.full((B, N), 1e10, jnp.float32)
            far[...] = jnp.zeros((B, 1), jnp.int32)

        f = far[...]
        o_ref[0] = f
        x = p_ref[0]
        y = p_ref[1]
        z = p_ref[2]
        iota = lax.broadcasted_iota(jnp.int32, (B, N), 1)
        sel = iota == f
        cx = jnp.sum(jnp.where(sel, x, 0.0), axis=1, keepdims=True)
        cy = jnp.sum(jnp.where(sel, y, 0.0), axis=1, keepdims=True)
        cz = jnp.sum(jnp.where(sel, z, 0.0), axis=1, keepdims=True)
        d = (x - cx) ** 2 + (y - cy) ** 2 + (z - cz) ** 2
        nd = jnp.minimum(dists[...], d)
        dists[...] = nd
        m = jnp.max(nd, axis=1, keepdims=True)
        cand = jnp.where(nd == m, iota, N)
        far[...] = jnp.min(cand, axis=1, keepdims=True)

    out = pl.pallas_call(
        kern,
        grid=(npoint,),
        in_specs=[pl.BlockSpec((3, B, N), lambda i: (0, 0, 0))],
        out_specs=pl.BlockSpec((1, B, 1), lambda i: (i, 0, 0)),
        out_shape=jax.ShapeDtypeStruct((npoint, B, 1), jnp.int32),
        scratch_shapes=[pltpu.VMEM((B, N), jnp.float32),
                        pltpu.VMEM((B, 1), jnp.int32)],
    )(pts_cbn)
    return out[:, :, 0].T


def _bf(x):
    return x.astype(jnp.bfloat16).astype(jnp.float32)


def _dist_parts(q, p):
    q0, q1, q2 = q[:, 0:1], q[:, 1:2], q[:, 2:3]
    p0, p1, p2 = p[0:1, :], p[1:2, :], p[2:3, :]
    cross = (_bf(q0) * _bf(p0) + _bf(q1) * _bf(p1) + _bf(q2) * _bf(p2))
    qn = q0 * q0 + q1 * q1 + q2 * q2
    pn = p0 * p0 + p1 * p1 + p2 * p2
    return (qn - 2.0 * cross) + pn


def _ballq(q_b, p_cn, r2, S):
    B, Q, _ = q_b.shape
    N = p_cn.shape[2]
    BQ = min(Q, 256)

    def kern(q_ref, p_ref, o_ref):
        b = pl.program_id(0)
        d = _dist_parts(q_ref[0], p_ref[0])
        iota = lax.broadcasted_iota(jnp.int32, (BQ, N), 1)
        keys = jnp.where(d <= r2, iota, iota + N)
        cols = []
        first = None
        for s in range(S):
            m = jnp.min(keys, axis=1, keepdims=True)
            if s == 0:
                first = jnp.where(m < N, m, 0)
                selc = first
            else:
                selc = jnp.where(m < N, m, first)
            cols.append(selc)
            keys = jnp.where(keys == m, 2 * N, keys)
        o_ref[0] = jnp.concatenate(cols, axis=1) + b * N

    return pl.pallas_call(
        kern,
        grid=(B, Q // BQ),
        in_specs=[pl.BlockSpec((1, BQ, 3), lambda b, j: (b, j, 0)),
                  pl.BlockSpec((1, 3, N), lambda b, j: (b, 0, 0))],
        out_specs=pl.BlockSpec((1, BQ, S), lambda b, j: (b, j, 0)),
        out_shape=jax.ShapeDtypeStruct((B, Q, S), jnp.int32),
    )(q_b, p_cn)


def _knn(q_b, p_cn, k):
    B, Q, _ = q_b.shape
    N = p_cn.shape[2]
    BQ = min(Q, 512)

    def kern(q_ref, p_ref, od_ref, oi_ref):
        b = pl.program_id(0)
        d = _dist_parts(q_ref[0], p_ref[0])
        iota = lax.broadcasted_iota(jnp.int32, (BQ, N), 1)
        dcols, icols = [], []
        for _ in range(k):
            m = jnp.min(d, axis=1, keepdims=True)
            cand = jnp.where(d == m, iota, N)
            selc = jnp.min(cand, axis=1, keepdims=True)
            dcols.append(m)
            icols.append(selc + b * N)
            d = jnp.where(iota == selc, jnp.float32(1e30), d)
        od_ref[0] = jnp.concatenate(dcols, axis=1)
        oi_ref[0] = jnp.concatenate(icols, axis=1)

    return pl.pallas_call(
        kern,
        grid=(B, Q // BQ),
        in_specs=[pl.BlockSpec((1, BQ, 3), lambda b, j: (b, j, 0)),
                  pl.BlockSpec((1, 3, N), lambda b, j: (b, 0, 0))],
        out_specs=[pl.BlockSpec((1, BQ, k), lambda b, j: (b, j, 0)),
                   pl.BlockSpec((1, BQ, k), lambda b, j: (b, j, 0))],
        out_shape=[jax.ShapeDtypeStruct((B, Q, k), jnp.float32),
                   jax.ShapeDtypeStruct((B, Q, k), jnp.int32)],
    )(q_b, p_cn)


def _mlp(x, layers, *, c=None, pre_relu=False, pool=None, S=1, wd=None,
         BM=2048):
    M, C0 = x.shape
    BM = min(BM, M)
    nblk = M // BM
    CL = layers[-1][0].shape[0] if layers else C0
    Mout, BMout = (M // S, BM // S) if pool else (M, BM)

    args = [x]
    in_specs = [pl.BlockSpec((BM, C0), lambda i: (i, 0))]
    if c is not None:
        args.append(c)
        in_specs.append(pl.BlockSpec((BM, C0), lambda i: (i, 0)))
    if wd is not None:
        args.append(wd)
        in_specs.append(pl.BlockSpec((BMout, S), lambda i: (i, 0)))
    for (W, b, _r) in layers:
        Wa = W.T if W.shape[1] <= 8 else W.astype(jnp.bfloat16)
        args.append(Wa)
        in_specs.append(pl.BlockSpec(Wa.shape, lambda i: (0, 0)))
        args.append(b.reshape(1, -1))
        in_specs.append(pl.BlockSpec((1, b.shape[0]), lambda i: (0, 0)))

    def kern(*refs):
        it = iter(refs)
        h = next(it)[...]
        if c is not None:
            h = h + next(it)[...]
        if pre_relu:
            h = jnp.maximum(h, 0.0)
        wd_v = next(it)[...] if wd is not None else None
        for (W, _b, r) in layers:
            Wv = next(it)[...]
            bv = next(it)[...]
            if W.shape[1] <= 8:
                acc = bv
                for cc in range(W.shape[1]):
                    acc = acc + h[:, cc:cc + 1] * Wv[cc:cc + 1, :]
                h = acc
            else:
                h = lax.dot_general(h.astype(jnp.bfloat16), Wv,
                                    (((1,), (1,)), ((), ())),
                                    preferred_element_type=jnp.float32) + bv
            if r:
                h = jnp.maximum(h, 0.0)
        o_ref = next(it)
        if pool == 'max':
            h = jnp.max(h.reshape(BMout, S, CL), axis=1)
        elif pool == 'wsum':
            w = 1.0 / jnp.maximum(wd_v, 1e-10)
            w = w / jnp.sum(w, axis=1, keepdims=True)
            h = jnp.sum(h.reshape(BMout, S, CL) * w[:, :, None], axis=1)
        o_ref[...] = h

    return pl.pallas_call(
        kern,
        grid=(nblk,),
        in_specs=in_specs,
        out_specs=pl.BlockSpec((BMout, CL), lambda i: (i, 0)),
        out_shape=jax.ShapeDtypeStruct((Mout, CL), jnp.float32),
    )(*args)


def _sc_gather_impl(table, idx):
    _T, C = table.shape
    M = idx.shape[0]
    info = plsc.get_sparse_core_info()
    NC = info.num_cores
    NW = NC * info.num_subcores
    bpw = M // NW
    CH = 128 if bpw % 128 == 0 else bpw
    nch = bpw // CH
    mesh = plsc.VectorSubcoreMesh(core_axis_name="c", subcore_axis_name="s")

    @functools.partial(
        pl.kernel, mesh=mesh,
        out_type=jax.ShapeDtypeStruct((M, C), jnp.float32),
        scratch_types=[pltpu.VMEM((CH,), jnp.int32),
                       pltpu.VMEM((CH, C), jnp.float32),
                       pltpu.SemaphoreType.DMA],
    )
    def k(table_hbm, idx_hbm, out_hbm, idx_v, rows_v, sem):
        wid = lax.axis_index("s") * NC + lax.axis_index("c")
        base = wid * bpw

        def chunk(off):
            pltpu.sync_copy(idx_hbm.at[pl.ds(off, CH)], idx_v)
            pltpu.async_copy(table_hbm.at[idx_v], rows_v, sem).wait()
            pltpu.sync_copy(rows_v, out_hbm.at[pl.ds(off, CH)])

        if nch == 1:
            chunk(base)
        else:
            def body(j, carry):
                chunk(base + j * CH)
                return carry
            lax.fori_loop(0, nch, body, 0)

    return k(table, idx)


def _gather_rows(table, idx):
    M = idx.shape[0]
    C = table.shape[1]
    Cp = ((C + 127) // 128) * 128
    if Cp != C:
        table = jnp.pad(table, ((0, 0), (0, Cp - C)))
    Mp = ((M + 255) // 256) * 256
    if Mp != M:
        idx = jnp.concatenate([idx, jnp.zeros((Mp - M,), jnp.int32)])
    out = _sc_gather_impl(table, idx)
    return out[:M, :C]


def _set_conv(pts_cn, feats, npoint, r2, S, params):
    B, _, N = pts_cn.shape
    (W1, b1), (W2, b2), (W3, b3) = params
    idx = _fps(jnp.transpose(pts_cn, (1, 0, 2)), npoint)
    gidx = (idx + jnp.arange(B, dtype=jnp.int32)[:, None] * N).reshape(-1)
    ptsT = jnp.transpose(pts_cn, (0, 2, 1))
    ptab = jnp.pad(ptsT.reshape(B * N, 3), ((0, 0), (0, 13)))
    new_flat = _gather_rows(ptab, gidx)[:, :3]
    new_b = new_flat.reshape(B, npoint, 3)
    nidx = _ballq(new_b, pts_cn, r2, S)
    src = jnp.concatenate([ptsT, feats], axis=-1).reshape(B * N, -1)
    A = _mlp(src, [(W1, jnp.zeros_like(b1), False)])
    cvec = _mlp(new_flat, [(-W1[:, :3], b1, False)])
    G = _gather_rows(A, nidx.reshape(-1))
    crep = jnp.repeat(cvec, S, axis=0)
    H = _mlp(G, [(W2, b2, True), (W3, b3, True)],
             c=crep, pre_relu=True, pool='max', S=S)
    return jnp.transpose(new_b, (0, 2, 1)), new_b, H.reshape(B, npoint, -1)


def _flow_embedding(p1_cn, p1_b, p2_cn, f1, f2, S, params):
    B, Q, _ = p1_b.shape
    N = p2_cn.shape[2]
    (W1, b1), (W2, b2), (W3, b3) = params
    Cf = f2.shape[-1]
    _, nidx = _knn(p1_b, p2_cn, S)
    src = jnp.concatenate([jnp.transpose(p2_cn, (0, 2, 1)), f2],
                          axis=-1).reshape(B * N, -1)
    A = _mlp(src, [(W1[:, :3 + Cf], jnp.zeros_like(b1), False)])
    qsrc = jnp.concatenate([p1_b, f1], axis=-1).reshape(B * Q, -1)
    Wc = jnp.concatenate([-W1[:, :3], W1[:, 3 + Cf:]], axis=1)
    cvec = _mlp(qsrc, [(Wc, b1, False)])
    G = _gather_rows(A, nidx.reshape(-1))
    crep = jnp.repeat(cvec, S, axis=0)
    H = _mlp(G, [(W2, b2, True), (W3, b3, True)],
             c=crep, pre_relu=True, pool='max', S=S)
    return H.reshape(B, Q, -1)


def _set_upconv(pc_cn, pf_b, fc, ff, S, params1, params2):
    B, Nf, _ = pf_b.shape
    Nc = pc_cn.shape[2]
    Cc = fc.shape[-1]
    _, nidx = _knn(pf_b, pc_cn, S)
    pf_flat = pf_b.reshape(B * Nf, 3)
    pcT = jnp.transpose(pc_cn, (0, 2, 1))
    if params1:
        (W1, b1), (W2, b2), (W3, b3) = params1
        src = jnp.concatenate([pcT, fc], axis=-1).reshape(B * Nc, -1)
        A = _mlp(src, [(W1, jnp.zeros_like(b1), False)])
        cvec = _mlp(pf_flat, [(-W1[:, :3], b1, False)])
        G = _gather_rows(A, nidx.reshape(-1))
        crep = jnp.repeat(cvec, S, axis=0)
        H = _mlp(G, [(W2, b2, True), (W3, b3, True)],
                 c=crep, pre_relu=True, pool='max', S=S)
        lay2 = params2
    else:
        tab = jnp.concatenate([jnp.pad(pcT, ((0, 0), (0, 0), (0, 13))), fc],
                              axis=-1).reshape(B * Nc, -1)
        G = _gather_rows(tab, nidx.reshape(-1))
        cpad = jnp.pad(-pf_flat, ((0, 0), (0, 13 + Cc)))
        crep = jnp.repeat(cpad, S, axis=0)
        H = _mlp(G, [], c=crep, pool='max', S=S)
        (W1, b1) = params2[0]
        Z = jnp.zeros((W1.shape[0], 13), W1.dtype)
        W1m = jnp.concatenate([W1[:, :3], Z, W1[:, 3:]], axis=1)
        lay2 = [(W1m, b1)] + list(params2[1:])
    g2 = jnp.concatenate([H, ff.reshape(B * Nf, -1)], axis=-1)
    out = _mlp(g2, [(W, b, True) for (W, b) in lay2])
    return out.reshape(B, Nf, -1)


def _feature_prop_cls(pc_cn, pf_b, fc, ff, params_fp, params_cls):
    B, Nf, _ = pf_b.shape
    Nc = pc_cn.shape[2]
    d, nidx = _knn(pf_b, pc_cn, 3)
    G = _gather_rows(fc.reshape(B * Nc, -1), nidx.reshape(-1))
    interp = _mlp(G, [], pool='wsum', S=3,
                  wd=d.reshape(B * Nf, 3), BM=1536)
    x = jnp.concatenate([interp, ff.reshape(B * Nf, -1)], axis=-1)
    (Wf1, bf1), (Wf2, bf2) = params_fp
    (Wc1, bc1), (Wc2, bc2) = params_cls
    out = _mlp(x, [(Wf1, bf1, True), (Wf2, bf2, True),
                   (Wc1, bc1, True), (Wc2, bc2, False)])
    return out.reshape(B, Nf, -1)


def kernel(points1, points2, features1, features2, params):
    B = points1.shape[0]
    f1 = jnp.transpose(features1, (0, 2, 1))
    f2 = jnp.transpose(features2, (0, 2, 1))

    pc = jnp.concatenate([points1, points2], axis=0)
    fc = jnp.concatenate([f1, f2], axis=0)
    pp_1cn, pp_1b, ff_1 = _set_conv(pc, fc, 1024, 0.25, 16, params['sc1'])
    pp_2cn, pp_2b, ff_2 = _set_conv(pp_1cn, ff_1, 256, 1.0, 16, params['sc2'])
    p1_1cn, p1_1b, f1_1 = pp_1cn[:B], pp_1b[:B], ff_1[:B]
    p1_2cn, p1_2b, f1_2 = pp_2cn[:B], pp_2b[:B], ff_2[:B]
    p2_2cn, f2_2 = pp_2cn[B:], ff_2[B:]

    emb = _flow_embedding(p1_2cn, p1_2b, p2_2cn, f1_2, f2_2, 64, params['fe'])

    p1_3cn, p1_3b, f1_3 = _set_conv(p1_2cn, emb, 64, 4.0, 8, params['sc3'])
    p1_4cn, _p1_4b, f1_4 = _set_conv(p1_3cn, f1_3, 16, 16.0, 8, params['sc4'])

    nf1_3 = _set_upconv(p1_4cn, p1_3b, f1_4, f1_3, 8,
                        params['up1_1'], params['up1_2'])
    nf1_2 = _set_upconv(p1_3cn, p1_2b, nf1_3,
                        jnp.concatenate([f1_2, emb], axis=-1), 8,
                        params['up2_1'], params['up2_2'])
    nf1_1 = _set_upconv(p1_2cn, p1_1b, nf1_2, f1_1, 8,
                        params['up3_1'], params['up3_2'])

    p1b = jnp.transpose(points1, (0, 2, 1))
    flow = _feature_prop_cls(p1_1cn, p1b, nf1_1, f1,
                             params['fp'], params['cls'])
    return jnp.transpose(flow, (0, 2, 1))


# speedup vs baseline: 1.4835x; 1.0285x over previous
import functools

import jax
import jax.numpy as jnp
from jax import lax
from jax.experimental import pallas as pl
from jax.experimental.pallas import tpu as pltpu
from jax.experimental.pallas import tpu_sc as plsc


def _fps(pts_cbn, npoint):
    _, B, N = pts_cbn.shape

    def kern(p_ref, o_ref, dists, far):
        i = pl.program_id(0)

        @pl.when(i == 0)
        def _():
            dists[...] = jnp.full((B, N), 1e10, jnp.float32)
            far[...] = jnp.zeros((B, 1), jnp.int32)

        f = far[...]
        o_ref[0] = f
        x = p_ref[0]
        y = p_ref[1]
        z = p_ref[2]
        iota = lax.broadcasted_iota(jnp.int32, (B, N), 1)
        sel = iota == f
        cx = jnp.sum(jnp.where(sel, x, 0.0), axis=1, keepdims=True)
        cy = jnp.sum(jnp.where(sel, y, 0.0), axis=1, keepdims=True)
        cz = jnp.sum(jnp.where(sel, z, 0.0), axis=1, keepdims=True)
        d = (x - cx) ** 2 + (y - cy) ** 2 + (z - cz) ** 2
        nd = jnp.minimum(dists[...], d)
        dists[...] = nd
        m = jnp.max(nd, axis=1, keepdims=True)
        cand = jnp.where(nd == m, iota, N)
        far[...] = jnp.min(cand, axis=1, keepdims=True)

    out = pl.pallas_call(
        kern,
        grid=(npoint,),
        in_specs=[pl.BlockSpec((3, B, N), lambda i: (0, 0, 0))],
        out_specs=pl.BlockSpec((1, B, 1), lambda i: (i, 0, 0)),
        out_shape=jax.ShapeDtypeStruct((npoint, B, 1), jnp.int32),
        scratch_shapes=[pltpu.VMEM((B, N), jnp.float32),
                        pltpu.VMEM((B, 1), jnp.int32)],
    )(pts_cbn)
    return out[:, :, 0].T


def _bf(x):
    return x.astype(jnp.bfloat16).astype(jnp.float32)


def _dist_parts(q, p):
    q0, q1, q2 = q[:, 0:1], q[:, 1:2], q[:, 2:3]
    p0, p1, p2 = p[0:1, :], p[1:2, :], p[2:3, :]
    cross = (_bf(q0) * _bf(p0) + _bf(q1) * _bf(p1) + _bf(q2) * _bf(p2))
    qn = q0 * q0 + q1 * q1 + q2 * q2
    pn = p0 * p0 + p1 * p1 + p2 * p2
    return (qn - 2.0 * cross) + pn


def _ballq(q_b, p_cn, r2, S):
    B, Q, _ = q_b.shape
    N = p_cn.shape[2]
    BQ = min(Q, 256)

    def kern(q_ref, p_ref, o_ref):
        b = pl.program_id(0)
        d = _dist_parts(q_ref[0], p_ref[0])
        iota = lax.broadcasted_iota(jnp.int32, (BQ, N), 1)
        keys = jnp.where(d <= r2, iota, iota + N)
        cols = []
        first = None
        for s in range(S):
            m = jnp.min(keys, axis=1, keepdims=True)
            if s == 0:
                first = jnp.where(m < N, m, 0)
                selc = first
            else:
                selc = jnp.where(m < N, m, first)
            cols.append(selc)
            keys = jnp.where(keys == m, 2 * N, keys)
        o_ref[0] = jnp.concatenate(cols, axis=1) + b * N

    return pl.pallas_call(
        kern,
        grid=(B, Q // BQ),
        in_specs=[pl.BlockSpec((1, BQ, 3), lambda b, j: (b, j, 0)),
                  pl.BlockSpec((1, 3, N), lambda b, j: (b, 0, 0))],
        out_specs=pl.BlockSpec((1, BQ, S), lambda b, j: (b, j, 0)),
        out_shape=jax.ShapeDtypeStruct((B, Q, S), jnp.int32),
    )(q_b, p_cn)


def _knn(q_b, p_cn, k):
    B, Q, _ = q_b.shape
    N = p_cn.shape[2]
    BQ = min(Q, 512)

    def kern(q_ref, p_ref, od_ref, oi_ref):
        b = pl.program_id(0)
        d = _dist_parts(q_ref[0], p_ref[0])
        iota = lax.broadcasted_iota(jnp.int32, (BQ, N), 1)
        dcols, icols = [], []
        for _ in range(k):
            m = jnp.min(d, axis=1, keepdims=True)
            cand = jnp.where(d == m, iota, N)
            selc = jnp.min(cand, axis=1, keepdims=True)
            dcols.append(m)
            icols.append(selc + b * N)
            d = jnp.where(iota == selc, jnp.float32(1e30), d)
        od_ref[0] = jnp.concatenate(dcols, axis=1)
        oi_ref[0] = jnp.concatenate(icols, axis=1)

    return pl.pallas_call(
        kern,
        grid=(B, Q // BQ),
        in_specs=[pl.BlockSpec((1, BQ, 3), lambda b, j: (b, j, 0)),
                  pl.BlockSpec((1, 3, N), lambda b, j: (b, 0, 0))],
        out_specs=[pl.BlockSpec((1, BQ, k), lambda b, j: (b, j, 0)),
                   pl.BlockSpec((1, BQ, k), lambda b, j: (b, j, 0))],
        out_shape=[jax.ShapeDtypeStruct((B, Q, k), jnp.float32),
                   jax.ShapeDtypeStruct((B, Q, k), jnp.int32)],
    )(q_b, p_cn)


def _mlp(x, layers, *, c=None, pre_relu=False, pool=None, S=1, wd=None,
         BM=2048):
    M, C0 = x.shape
    BM = min(BM, M)
    nblk = M // BM
    CL = layers[-1][0].shape[0] if layers else C0
    Mout, BMout = (M // S, BM // S) if pool else (M, BM)

    args = [x]
    in_specs = [pl.BlockSpec((BM, C0), lambda i: (i, 0))]
    if c is not None:
        args.append(c)
        in_specs.append(pl.BlockSpec((BM, C0), lambda i: (i, 0)))
    if wd is not None:
        args.append(wd)
        in_specs.append(pl.BlockSpec((BMout, S), lambda i: (i, 0)))
    for (W, b, _r) in layers:
        Wa = W.T if W.shape[1] <= 8 else W.astype(jnp.bfloat16)
        args.append(Wa)
        in_specs.append(pl.BlockSpec(Wa.shape, lambda i: (0, 0)))
        args.append(b.reshape(1, -1))
        in_specs.append(pl.BlockSpec((1, b.shape[0]), lambda i: (0, 0)))

    def kern(*refs):
        it = iter(refs)
        h = next(it)[...]
        if c is not None:
            h = h + next(it)[...]
        if pre_relu:
            h = jnp.maximum(h, 0.0)
        wd_v = next(it)[...] if wd is not None else None
        for (W, _b, r) in layers:
            Wv = next(it)[...]
            bv = next(it)[...]
            if W.shape[1] <= 8:
                acc = bv
                for cc in range(W.shape[1]):
                    acc = acc + h[:, cc:cc + 1] * Wv[cc:cc + 1, :]
                h = acc
            else:
                h = lax.dot_general(h.astype(jnp.bfloat16), Wv,
                                    (((1,), (1,)), ((), ())),
                                    preferred_element_type=jnp.float32) + bv
            if r:
                h = jnp.maximum(h, 0.0)
        o_ref = next(it)
        if pool == 'max':
            h = jnp.max(h.reshape(BMout, S, CL), axis=1)
        elif pool == 'wsum':
            w = 1.0 / jnp.maximum(wd_v, 1e-10)
            w = w / jnp.sum(w, axis=1, keepdims=True)
            h = jnp.sum(h.reshape(BMout, S, CL) * w[:, :, None], axis=1)
        o_ref[...] = h

    return pl.pallas_call(
        kern,
        grid=(nblk,),
        in_specs=in_specs,
        out_specs=pl.BlockSpec((BMout, CL), lambda i: (i, 0)),
        out_shape=jax.ShapeDtypeStruct((Mout, CL), jnp.float32),
    )(*args)


def _sc_gather_impl(table, idx):
    _T, C = table.shape
    M = idx.shape[0]
    info = plsc.get_sparse_core_info()
    NC = info.num_cores
    NW = NC * info.num_subcores
    bpw = M // NW
    CH = 128 if bpw % 128 == 0 else bpw
    while 2 * CH * C * 4 + bpw * 4 > 450_000:
        CH //= 2
    nch = bpw // CH
    mesh = plsc.VectorSubcoreMesh(core_axis_name="c", subcore_axis_name="s")

    @functools.partial(
        pl.kernel, mesh=mesh,
        out_type=jax.ShapeDtypeStruct((M, C), jnp.float32),
        scratch_types=[pltpu.VMEM((bpw,), jnp.int32),
                       pltpu.VMEM((CH, C), jnp.float32),
                       pltpu.VMEM((CH, C), jnp.float32),
                       pltpu.SemaphoreType.DMA,
                       pltpu.SemaphoreType.DMA],
    )
    def k(table_hbm, idx_hbm, out_hbm, idx_v, r0, r1, s0, s1):
        wid = lax.axis_index("s") * NC + lax.axis_index("c")
        base = wid * bpw
        pltpu.sync_copy(idx_hbm.at[pl.ds(base, bpw)], idx_v)

        def gcopy(j, rv, sv):
            return pltpu.make_async_copy(
                table_hbm.at[idx_v.at[pl.ds(j * CH, CH)]], rv, sv)

        if nch == 1:
            gcopy(0, r0, s0).start()
            gcopy(0, r0, s0).wait()
            pltpu.sync_copy(r0, out_hbm.at[pl.ds(base, CH)])
        else:
            gcopy(0, r0, s0).start()

            def body(j, carry):
                even = j % 2 == 0

                @pl.when((j + 1 < nch) & even)
                def _():
                    gcopy(j + 1, r1, s1).start()

                @pl.when((j + 1 < nch) & jnp.logical_not(even))
                def _():
                    gcopy(j + 1, r0, s0).start()

                @pl.when(even)
                def _():
                    gcopy(j, r0, s0).wait()
                    pltpu.sync_copy(r0, out_hbm.at[pl.ds(base + j * CH, CH)])

                @pl.when(jnp.logical_not(even))
                def _():
                    gcopy(j, r1, s1).wait()
                    pltpu.sync_copy(r1, out_hbm.at[pl.ds(base + j * CH, CH)])

                return carry

            lax.fori_loop(0, nch, body, 0)

    return k(table, idx)


def _gather_rows(table, idx):
    M = idx.shape[0]
    C = table.shape[1]
    Cp = ((C + 127) // 128) * 128
    if Cp != C:
        table = jnp.pad(table, ((0, 0), (0, Cp - C)))
    Mp = ((M + 255) // 256) * 256
    if Mp != M:
        idx = jnp.concatenate([idx, jnp.zeros((Mp - M,), jnp.int32)])
    out = _sc_gather_impl(table, idx)
    return out[:M, :C]


def _set_conv(pts_cn, feats, npoint, r2, S, params):
    B, _, N = pts_cn.shape
    (W1, b1), (W2, b2), (W3, b3) = params
    idx = _fps(jnp.transpose(pts_cn, (1, 0, 2)), npoint)
    gidx = (idx + jnp.arange(B, dtype=jnp.int32)[:, None] * N).reshape(-1)
    ptsT = jnp.transpose(pts_cn, (0, 2, 1))
    ptab = jnp.pad(ptsT.reshape(B * N, 3), ((0, 0), (0, 13)))
    new_flat = _gather_rows(ptab, gidx)[:, :3]
    new_b = new_flat.reshape(B, npoint, 3)
    nidx = _ballq(new_b, pts_cn, r2, S)
    src = jnp.concatenate([ptsT, feats], axis=-1).reshape(B * N, -1)
    A = _mlp(src, [(W1, jnp.zeros_like(b1), False)])
    cvec = _mlp(new_flat, [(-W1[:, :3], b1, False)])
    G = _gather_rows(A, nidx.reshape(-1))
    crep = jnp.repeat(cvec, S, axis=0)
    H = _mlp(G, [(W2, b2, True), (W3, b3, True)],
             c=crep, pre_relu=True, pool='max', S=S)
    return jnp.transpose(new_b, (0, 2, 1)), new_b, H.reshape(B, npoint, -1)


def _flow_embedding(p1_cn, p1_b, p2_cn, f1, f2, S, params):
    B, Q, _ = p1_b.shape
    N = p2_cn.shape[2]
    (W1, b1), (W2, b2), (W3, b3) = params
    Cf = f2.shape[-1]
    _, nidx = _knn(p1_b, p2_cn, S)
    src = jnp.concatenate([jnp.transpose(p2_cn, (0, 2, 1)), f2],
                          axis=-1).reshape(B * N, -1)
    A = _mlp(src, [(W1[:, :3 + Cf], jnp.zeros_like(b1), False)])
    qsrc = jnp.concatenate([p1_b, f1], axis=-1).reshape(B * Q, -1)
    Wc = jnp.concatenate([-W1[:, :3], W1[:, 3 + Cf:]], axis=1)
    cvec = _mlp(qsrc, [(Wc, b1, False)])
    G = _gather_rows(A, nidx.reshape(-1))
    crep = jnp.repeat(cvec, S, axis=0)
    H = _mlp(G, [(W2, b2, True), (W3, b3, True)],
             c=crep, pre_relu=True, pool='max', S=S)
    return H.reshape(B, Q, -1)


def _set_upconv(pc_cn, pf_b, fc, ff, S, params1, params2):
    B, Nf, _ = pf_b.shape
    Nc = pc_cn.shape[2]
    Cc = fc.shape[-1]
    _, nidx = _knn(pf_b, pc_cn, S)
    pf_flat = pf_b.reshape(B * Nf, 3)
    pcT = jnp.transpose(pc_cn, (0, 2, 1))
    if params1:
        (W1, b1), (W2, b2), (W3, b3) = params1
        src = jnp.concatenate([pcT, fc], axis=-1).reshape(B * Nc, -1)
        A = _mlp(src, [(W1, jnp.zeros_like(b1), False)])
        cvec = _mlp(pf_flat, [(-W1[:, :3], b1, False)])
        G = _gather_rows(A, nidx.reshape(-1))
        crep = jnp.repeat(cvec, S, axis=0)
        H = _mlp(G, [(W2, b2, True), (W3, b3, True)],
                 c=crep, pre_relu=True, pool='max', S=S)
        lay2 = params2
    else:
        tab = jnp.concatenate([jnp.pad(pcT, ((0, 0), (0, 0), (0, 13))), fc],
                              axis=-1).reshape(B * Nc, -1)
        G = _gather_rows(tab, nidx.reshape(-1))
        cpad = jnp.pad(-pf_flat, ((0, 0), (0, 13 + Cc)))
        crep = jnp.repeat(cpad, S, axis=0)
        H = _mlp(G, [], c=crep, pool='max', S=S)
        (W1, b1) = params2[0]
        Z = jnp.zeros((W1.shape[0], 13), W1.dtype)
        W1m = jnp.concatenate([W1[:, :3], Z, W1[:, 3:]], axis=1)
        lay2 = [(W1m, b1)] + list(params2[1:])
    g2 = jnp.concatenate([H, ff.reshape(B * Nf, -1)], axis=-1)
    out = _mlp(g2, [(W, b, True) for (W, b) in lay2])
    return out.reshape(B, Nf, -1)


def _feature_prop_cls(pc_cn, pf_b, fc, ff, params_fp, params_cls):
    B, Nf, _ = pf_b.shape
    Nc = pc_cn.shape[2]
    d, nidx = _knn(pf_b, pc_cn, 3)
    G = _gather_rows(fc.reshape(B * Nc, -1), nidx.reshape(-1))
    interp = _mlp(G, [], pool='wsum', S=3,
                  wd=d.reshape(B * Nf, 3), BM=1536)
    x = jnp.concatenate([interp, ff.reshape(B * Nf, -1)], axis=-1)
    (Wf1, bf1), (Wf2, bf2) = params_fp
    (Wc1, bc1), (Wc2, bc2) = params_cls
    out = _mlp(x, [(Wf1, bf1, True), (Wf2, bf2, True),
                   (Wc1, bc1, True), (Wc2, bc2, False)])
    return out.reshape(B, Nf, -1)


def kernel(points1, points2, features1, features2, params):
    B = points1.shape[0]
    f1 = jnp.transpose(features1, (0, 2, 1))
    f2 = jnp.transpose(features2, (0, 2, 1))

    pc = jnp.concatenate([points1, points2], axis=0)
    fc = jnp.concatenate([f1, f2], axis=0)
    pp_1cn, pp_1b, ff_1 = _set_conv(pc, fc, 1024, 0.25, 16, params['sc1'])
    pp_2cn, pp_2b, ff_2 = _set_conv(pp_1cn, ff_1, 256, 1.0, 16, params['sc2'])
    p1_1cn, p1_1b, f1_1 = pp_1cn[:B], pp_1b[:B], ff_1[:B]
    p1_2cn, p1_2b, f1_2 = pp_2cn[:B], pp_2b[:B], ff_2[:B]
    p2_2cn, f2_2 = pp_2cn[B:], ff_2[B:]

    emb = _flow_embedding(p1_2cn, p1_2b, p2_2cn, f1_2, f2_2, 64, params['fe'])

    p1_3cn, p1_3b, f1_3 = _set_conv(p1_2cn, emb, 64, 4.0, 8, params['sc3'])
    p1_4cn, _p1_4b, f1_4 = _set_conv(p1_3cn, f1_3, 16, 16.0, 8, params['sc4'])

    nf1_3 = _set_upconv(p1_4cn, p1_3b, f1_4, f1_3, 8,
                        params['up1_1'], params['up1_2'])
    nf1_2 = _set_upconv(p1_3cn, p1_2b, nf1_3,
                        jnp.concatenate([f1_2, emb], axis=-1), 8,
                        params['up2_1'], params['up2_2'])
    nf1_1 = _set_upconv(p1_2cn, p1_1b, nf1_2, f1_1, 8,
                        params['up3_1'], params['up3_2'])

    p1b = jnp.transpose(points1, (0, 2, 1))
    flow = _feature_prop_cls(p1_1cn, p1b, nf1_1, f1,
                             params['fp'], params['cls'])
    return jnp.transpose(flow, (0, 2, 1))


# speedup vs baseline: 1.4903x; 1.0046x over previous
import functools

import jax
import jax.numpy as jnp
from jax import lax
from jax.experimental import pallas as pl
from jax.experimental.pallas import tpu as pltpu
from jax.experimental.pallas import tpu_sc as plsc


def _fps(pts_cbn, npoint):
    _, B, N = pts_cbn.shape

    def kern(p_ref, o_ref, dists, far):
        i = pl.program_id(0)

        @pl.when(i == 0)
        def _():
            dists[...] = jnp.full((B, N), 1e10, jnp.float32)
            far[...] = jnp.zeros((B, 1), jnp.int32)

        f = far[...]
        o_ref[0] = f
        x = p_ref[0]
        y = p_ref[1]
        z = p_ref[2]
        iota = lax.broadcasted_iota(jnp.int32, (B, N), 1)
        sel = iota == f
        cx = jnp.sum(jnp.where(sel, x, 0.0), axis=1, keepdims=True)
        cy = jnp.sum(jnp.where(sel, y, 0.0), axis=1, keepdims=True)
        cz = jnp.sum(jnp.where(sel, z, 0.0), axis=1, keepdims=True)
        d = (x - cx) ** 2 + (y - cy) ** 2 + (z - cz) ** 2
        nd = jnp.minimum(dists[...], d)
        dists[...] = nd
        m = jnp.max(nd, axis=1, keepdims=True)
        cand = jnp.where(nd == m, iota, N)
        far[...] = jnp.min(cand, axis=1, keepdims=True)

    out = pl.pallas_call(
        kern,
        grid=(npoint,),
        in_specs=[pl.BlockSpec((3, B, N), lambda i: (0, 0, 0))],
        out_specs=pl.BlockSpec((1, B, 1), lambda i: (i, 0, 0)),
        out_shape=jax.ShapeDtypeStruct((npoint, B, 1), jnp.int32),
        scratch_shapes=[pltpu.VMEM((B, N), jnp.float32),
                        pltpu.VMEM((B, 1), jnp.int32)],
    )(pts_cbn)
    return out[:, :, 0].T


def _bf(x):
    return x.astype(jnp.bfloat16).astype(jnp.float32)


def _dist_parts(q, p):
    q0, q1, q2 = q[:, 0:1], q[:, 1:2], q[:, 2:3]
    p0, p1, p2 = p[0:1, :], p[1:2, :], p[2:3, :]
    cross = (_bf(q0) * _bf(p0) + _bf(q1) * _bf(p1) + _bf(q2) * _bf(p2))
    qn = q0 * q0 + q1 * q1 + q2 * q2
    pn = p0 * p0 + p1 * p1 + p2 * p2
    return (qn - 2.0 * cross) + pn


def _ballq(q_b, p_cn, r2, S):
    B, Q, _ = q_b.shape
    N = p_cn.shape[2]
    BQ = min(Q, 256)

    def kern(q_ref, p_ref, o_ref):
        b = pl.program_id(0)
        d = _dist_parts(q_ref[0], p_ref[0])
        iota = lax.broadcasted_iota(jnp.int32, (BQ, N), 1)
        keys = jnp.where(d <= r2, iota, iota + N)
        cols = []
        first = None
        for s in range(S):
            m = jnp.min(keys, axis=1, keepdims=True)
            if s == 0:
                first = jnp.where(m < N, m, 0)
                selc = first
            else:
                selc = jnp.where(m < N, m, first)
            cols.append(selc)
            keys = jnp.where(keys == m, 2 * N, keys)
        o_ref[0] = jnp.concatenate(cols, axis=1) + b * N

    return pl.pallas_call(
        kern,
        grid=(B, Q // BQ),
        in_specs=[pl.BlockSpec((1, BQ, 3), lambda b, j: (b, j, 0)),
                  pl.BlockSpec((1, 3, N), lambda b, j: (b, 0, 0))],
        out_specs=pl.BlockSpec((1, BQ, S), lambda b, j: (b, j, 0)),
        out_shape=jax.ShapeDtypeStruct((B, Q, S), jnp.int32),
    )(q_b, p_cn)


def _knn(q_b, p_cn, k):
    B, Q, _ = q_b.shape
    N = p_cn.shape[2]
    BQ = min(Q, 512)

    def kern(q_ref, p_ref, od_ref, oi_ref):
        b = pl.program_id(0)
        d = _dist_parts(q_ref[0], p_ref[0])
        iota = lax.broadcasted_iota(jnp.int32, (BQ, N), 1)
        dcols, icols = [], []
        for _ in range(k):
            m = jnp.min(d, axis=1, keepdims=True)
            cand = jnp.where(d == m, iota, N)
            selc = jnp.min(cand, axis=1, keepdims=True)
            dcols.append(m)
            icols.append(selc + b * N)
            d = jnp.where(iota == selc, jnp.float32(1e30), d)
        od_ref[0] = jnp.concatenate(dcols, axis=1)
        oi_ref[0] = jnp.concatenate(icols, axis=1)

    return pl.pallas_call(
        kern,
        grid=(B, Q // BQ),
        in_specs=[pl.BlockSpec((1, BQ, 3), lambda b, j: (b, j, 0)),
                  pl.BlockSpec((1, 3, N), lambda b, j: (b, 0, 0))],
        out_specs=[pl.BlockSpec((1, BQ, k), lambda b, j: (b, j, 0)),
                   pl.BlockSpec((1, BQ, k), lambda b, j: (b, j, 0))],
        out_shape=[jax.ShapeDtypeStruct((B, Q, k), jnp.float32),
                   jax.ShapeDtypeStruct((B, Q, k), jnp.int32)],
    )(q_b, p_cn)


def _mlp(x, layers, *, c=None, pre_relu=False, pool=None, S=1, wd=None,
         BM=2048):
    M, C0 = x.shape
    BM = min(BM, M)
    nblk = M // BM
    CL = layers[-1][0].shape[0] if layers else C0
    Mout, BMout = (M // S, BM // S) if pool else (M, BM)

    args = [x]
    in_specs = [pl.BlockSpec((BM, C0), lambda i: (i, 0))]
    if c is not None:
        args.append(c)
        in_specs.append(pl.BlockSpec((BM // S, C0), lambda i: (i, 0)))
    if wd is not None:
        args.append(wd)
        in_specs.append(pl.BlockSpec((BMout, S), lambda i: (i, 0)))
    for (W, b, _r) in layers:
        Wa = W.T if W.shape[1] <= 8 else W.astype(jnp.bfloat16)
        args.append(Wa)
        in_specs.append(pl.BlockSpec(Wa.shape, lambda i: (0, 0)))
        args.append(b.reshape(1, -1))
        in_specs.append(pl.BlockSpec((1, b.shape[0]), lambda i: (0, 0)))

    def kern(*refs):
        it = iter(refs)
        h = next(it)[...]
        if c is not None:
            cb = next(it)[...]
            h = (h.reshape(BM // S, S, C0) + cb[:, None, :]).reshape(BM, C0)
        if pre_relu:
            h = jnp.maximum(h, 0.0)
        wd_v = next(it)[...] if wd is not None else None
        for (W, _b, r) in layers:
            Wv = next(it)[...]
            bv = next(it)[...]
            if W.shape[1] <= 8:
                acc = bv
                for cc in range(W.shape[1]):
                    acc = acc + h[:, cc:cc + 1] * Wv[cc:cc + 1, :]
                h = acc
            else:
                h = lax.dot_general(h.astype(jnp.bfloat16), Wv,
                                    (((1,), (1,)), ((), ())),
                                    preferred_element_type=jnp.float32) + bv
            if r:
                h = jnp.maximum(h, 0.0)
        o_ref = next(it)
        if pool == 'max':
            h = jnp.max(h.reshape(BMout, S, CL), axis=1)
        elif pool == 'wsum':
            w = 1.0 / jnp.maximum(wd_v, 1e-10)
            w = w / jnp.sum(w, axis=1, keepdims=True)
            h = jnp.sum(h.reshape(BMout, S, CL) * w[:, :, None], axis=1)
        o_ref[...] = h

    return pl.pallas_call(
        kern,
        grid=(nblk,),
        in_specs=in_specs,
        out_specs=pl.BlockSpec((BMout, CL), lambda i: (i, 0)),
        out_shape=jax.ShapeDtypeStruct((Mout, CL), jnp.float32),
    )(*args)


def _sc_gather_impl(table, idx, tc_tiling):
    _T, C = table.shape
    M = idx.shape[0]
    info = plsc.get_sparse_core_info()
    NC = info.num_cores
    NW = NC * info.num_subcores
    bpw = M // NW
    CH = 128 if bpw % 128 == 0 else bpw
    while 2 * CH * C * 4 + bpw * 4 > 450_000:
        CH //= 2
    nch = bpw // CH
    mesh = plsc.VectorSubcoreMesh(core_axis_name="c", subcore_axis_name="s")

    @functools.partial(
        pl.kernel, mesh=mesh,
        compiler_params=pltpu.CompilerParams(use_tc_tiling_on_sc=tc_tiling),
        out_type=jax.ShapeDtypeStruct((M, C), jnp.float32),
        scratch_types=[pltpu.VMEM((bpw,), jnp.int32),
                       pltpu.VMEM((CH, C), jnp.float32),
                       pltpu.VMEM((CH, C), jnp.float32),
                       pltpu.SemaphoreType.DMA,
                       pltpu.SemaphoreType.DMA],
    )
    def k(table_hbm, idx_hbm, out_hbm, idx_v, r0, r1, s0, s1):
        wid = lax.axis_index("s") * NC + lax.axis_index("c")
        base = wid * bpw
        pltpu.sync_copy(idx_hbm.at[pl.ds(base, bpw)], idx_v)

        def gcopy(j, rv, sv):
            return pltpu.make_async_copy(
                table_hbm.at[idx_v.at[pl.ds(j * CH, CH)]], rv, sv)

        if nch == 1:
            gcopy(0, r0, s0).start()
            gcopy(0, r0, s0).wait()
            pltpu.sync_copy(r0, out_hbm.at[pl.ds(base, CH)])
        else:
            gcopy(0, r0, s0).start()

            def body(j, carry):
                even = j % 2 == 0

                @pl.when((j + 1 < nch) & even)
                def _():
                    gcopy(j + 1, r1, s1).start()

                @pl.when((j + 1 < nch) & jnp.logical_not(even))
                def _():
                    gcopy(j + 1, r0, s0).start()

                @pl.when(even)
                def _():
                    gcopy(j, r0, s0).wait()
                    pltpu.sync_copy(r0, out_hbm.at[pl.ds(base + j * CH, CH)])

                @pl.when(jnp.logical_not(even))
                def _():
                    gcopy(j, r1, s1).wait()
                    pltpu.sync_copy(r1, out_hbm.at[pl.ds(base + j * CH, CH)])

                return carry

            lax.fori_loop(0, nch, body, 0)

    return k(table, idx)


def _gather_rows(table, idx):
    M = idx.shape[0]
    C = table.shape[1]
    tc_tiling = C > 64
    align = 128 if tc_tiling else 16
    Cp = ((C + align - 1) // align) * align
    if Cp != C:
        table = jnp.pad(table, ((0, 0), (0, Cp - C)))
    Mp = ((M + 255) // 256) * 256
    if Mp != M:
        idx = jnp.concatenate([idx, jnp.zeros((Mp - M,), jnp.int32)])
    out = _sc_gather_impl(table, idx, tc_tiling)
    return out[:M, :C]


def _set_conv(pts_cn, feats, npoint, r2, S, params):
    B, _, N = pts_cn.shape
    (W1, b1), (W2, b2), (W3, b3) = params
    idx = _fps(jnp.transpose(pts_cn, (1, 0, 2)), npoint)
    gidx = (idx + jnp.arange(B, dtype=jnp.int32)[:, None] * N).reshape(-1)
    ptsT = jnp.transpose(pts_cn, (0, 2, 1))
    ptab = jnp.pad(ptsT.reshape(B * N, 3), ((0, 0), (0, 13)))
    new_flat = _gather_rows(ptab, gidx)[:, :3]
    new_b = new_flat.reshape(B, npoint, 3)
    nidx = _ballq(new_b, pts_cn, r2, S)
    src = jnp.concatenate([ptsT, feats], axis=-1).reshape(B * N, -1)
    A = _mlp(src, [(W1, jnp.zeros_like(b1), False)])
    cvec = _mlp(new_flat, [(-W1[:, :3], b1, False)])
    G = _gather_rows(A, nidx.reshape(-1))
    H = _mlp(G, [(W2, b2, True), (W3, b3, True)],
             c=cvec, pre_relu=True, pool='max', S=S)
    return jnp.transpose(new_b, (0, 2, 1)), new_b, H.reshape(B, npoint, -1)


def _flow_embedding(p1_cn, p1_b, p2_cn, f1, f2, S, params):
    B, Q, _ = p1_b.shape
    N = p2_cn.shape[2]
    (W1, b1), (W2, b2), (W3, b3) = params
    Cf = f2.shape[-1]
    _, nidx = _knn(p1_b, p2_cn, S)
    src = jnp.concatenate([jnp.transpose(p2_cn, (0, 2, 1)), f2],
                          axis=-1).reshape(B * N, -1)
    A = _mlp(src, [(W1[:, :3 + Cf], jnp.zeros_like(b1), False)])
    qsrc = jnp.concatenate([p1_b, f1], axis=-1).reshape(B * Q, -1)
    Wc = jnp.concatenate([-W1[:, :3], W1[:, 3 + Cf:]], axis=1)
    cvec = _mlp(qsrc, [(Wc, b1, False)])
    G = _gather_rows(A, nidx.reshape(-1))
    H = _mlp(G, [(W2, b2, True), (W3, b3, True)],
             c=cvec, pre_relu=True, pool='max', S=S)
    return H.reshape(B, Q, -1)


def _set_upconv(pc_cn, pf_b, fc, ff, S, params1, params2):
    B, Nf, _ = pf_b.shape
    Nc = pc_cn.shape[2]
    Cc = fc.shape[-1]
    _, nidx = _knn(pf_b, pc_cn, S)
    pf_flat = pf_b.reshape(B * Nf, 3)
    pcT = jnp.transpose(pc_cn, (0, 2, 1))
    if params1:
        (W1, b1), (W2, b2), (W3, b3) = params1
        src = jnp.concatenate([pcT, fc], axis=-1).reshape(B * Nc, -1)
        A = _mlp(src, [(W1, jnp.zeros_like(b1), False)])
        cvec = _mlp(pf_flat, [(-W1[:, :3], b1, False)])
        G = _gather_rows(A, nidx.reshape(-1))
        H = _mlp(G, [(W2, b2, True), (W3, b3, True)],
                 c=cvec, pre_relu=True, pool='max', S=S)
        lay2 = params2
    else:
        tab = jnp.concatenate([jnp.pad(pcT, ((0, 0), (0, 0), (0, 13))), fc],
                              axis=-1).reshape(B * Nc, -1)
        G = _gather_rows(tab, nidx.reshape(-1))
        cpad = jnp.pad(-pf_flat, ((0, 0), (0, 13 + Cc)))
        H = _mlp(G, [], c=cpad, pool='max', S=S)
        (W1, b1) = params2[0]
        Z = jnp.zeros((W1.shape[0], 13), W1.dtype)
        W1m = jnp.concatenate([W1[:, :3], Z, W1[:, 3:]], axis=1)
        lay2 = [(W1m, b1)] + list(params2[1:])
    g2 = jnp.concatenate([H, ff.reshape(B * Nf, -1)], axis=-1)
    out = _mlp(g2, [(W, b, True) for (W, b) in lay2])
    return out.reshape(B, Nf, -1)


def _feature_prop_cls(pc_cn, pf_b, fc, ff, params_fp, params_cls):
    B, Nf, _ = pf_b.shape
    Nc = pc_cn.shape[2]
    d, nidx = _knn(pf_b, pc_cn, 3)
    G = _gather_rows(fc.reshape(B * Nc, -1), nidx.reshape(-1))
    interp = _mlp(G, [], pool='wsum', S=3,
                  wd=d.reshape(B * Nf, 3), BM=1536)
    x = jnp.concatenate([interp, ff.reshape(B * Nf, -1)], axis=-1)
    (Wf1, bf1), (Wf2, bf2) = params_fp
    (Wc1, bc1), (Wc2, bc2) = params_cls
    out = _mlp(x, [(Wf1, bf1, True), (Wf2, bf2, True),
                   (Wc1, bc1, True), (Wc2, bc2, False)])
    return out.reshape(B, Nf, -1)


def kernel(points1, points2, features1, features2, params):
    B = points1.shape[0]
    f1 = jnp.transpose(features1, (0, 2, 1))
    f2 = jnp.transpose(features2, (0, 2, 1))

    pc = jnp.concatenate([points1, points2], axis=0)
    fc = jnp.concatenate([f1, f2], axis=0)
    pp_1cn, pp_1b, ff_1 = _set_conv(pc, fc, 1024, 0.25, 16, params['sc1'])
    pp_2cn, pp_2b, ff_2 = _set_conv(pp_1cn, ff_1, 256, 1.0, 16, params['sc2'])
    p1_1cn, p1_1b, f1_1 = pp_1cn[:B], pp_1b[:B], ff_1[:B]
    p1_2cn, p1_2b, f1_2 = pp_2cn[:B], pp_2b[:B], ff_2[:B]
    p2_2cn, f2_2 = pp_2cn[B:], ff_2[B:]

    emb = _flow_embedding(p1_2cn, p1_2b, p2_2cn, f1_2, f2_2, 64, params['fe'])

    p1_3cn, p1_3b, f1_3 = _set_conv(p1_2cn, emb, 64, 4.0, 8, params['sc3'])
    p1_4cn, _p1_4b, f1_4 = _set_conv(p1_3cn, f1_3, 16, 16.0, 8, params['sc4'])

    nf1_3 = _set_upconv(p1_4cn, p1_3b, f1_4, f1_3, 8,
                        params['up1_1'], params['up1_2'])
    nf1_2 = _set_upconv(p1_3cn, p1_2b, nf1_3,
                        jnp.concatenate([f1_2, emb], axis=-1), 8,
                        params['up2_1'], params['up2_2'])
    nf1_1 = _set_upconv(p1_2cn, p1_1b, nf1_2, f1_1, 8,
                        params['up3_1'], params['up3_2'])

    p1b = jnp.transpose(points1, (0, 2, 1))
    flow = _feature_prop_cls(p1_1cn, p1b, nf1_1, f1,
                             params['fp'], params['cls'])
    return jnp.transpose(flow, (0, 2, 1))
